# baseline (speedup 1.0000x reference)
import functools

import jax
import jax.numpy as jnp
from jax import lax
from jax.experimental import pallas as pl
from jax.experimental.pallas import tpu as pltpu
from jax.experimental.pallas import tpu_sc as plsc

B = 16384
D = 64
NROWS = 100000

NC = 2
NS = 16
NW = NC * NS
BPW = B // NW


PACK_C = 16640
PACK_GRID = (NROWS + PACK_C - 1) // PACK_C


def _pack_body(gt, mt, eye_top, eye_bot, out):
    dn = (((0,), (0,)), ((), ()))
    out[...] = (
        lax.dot_general(gt[...], eye_top[...], dn,
                        preferred_element_type=jnp.float32)
        + lax.dot_general(mt[...], eye_bot[...], dn,
                          preferred_element_type=jnp.float32))


def _pack_pair(gt_T, mt_T, eye_top, eye_bot):
    in_spec = pl.BlockSpec((D, PACK_C), lambda i: (0, i))
    eye_spec = pl.BlockSpec((D, 2 * D), lambda i: (0, 0))
    return pl.pallas_call(
        _pack_body,
        grid=(PACK_GRID,),
        in_specs=[in_spec, in_spec, eye_spec, eye_spec],
        out_specs=pl.BlockSpec((PACK_C, 2 * D), lambda i: (i, 0)),
        out_shape=jax.ShapeDtypeStruct((NROWS, 2 * D), jnp.float32),
    )(gt_T, mt_T, eye_top, eye_bot)


HBW = BPW // 2


@functools.lru_cache(maxsize=1)
def _make_sc_gather1():
    mesh = plsc.VectorSubcoreMesh(
        core_axis_name="c", subcore_axis_name="s",
        num_cores=NC, num_subcores=NS)

    @functools.partial(
        pl.kernel,
        out_type=jax.ShapeDtypeStruct((B, 2 * D), jnp.float32),
        mesh=mesh,
        scratch_types=[
            pltpu.VMEM((HBW,), jnp.int32),
            pltpu.VMEM((HBW,), jnp.int32),
            pltpu.VMEM((HBW, 2 * D), jnp.float32),
            pltpu.VMEM((HBW, 2 * D), jnp.float32),
            pltpu.SemaphoreType.DMA,
            pltpu.SemaphoreType.DMA,
        ],
        compiler_params=pltpu.CompilerParams(use_tc_tiling_on_sc=True),
    )
    def sc_gather1(t_hbm, ids, out, idx0, idx1, buf0, buf1, sem0, sem1):
        wid = lax.axis_index("s") * NC + lax.axis_index("c")
        base = wid * BPW
        pltpu.sync_copy(ids.at[pl.ds(base, HBW)], idx0)
        pltpu.sync_copy(ids.at[pl.ds(base + HBW, HBW)], idx1)
        cp0 = pltpu.async_copy(t_hbm.at[idx0], buf0, sem0)
        cp1 = pltpu.async_copy(t_hbm.at[idx1], buf1, sem1)
        cp0.wait()
        pltpu.sync_copy(buf0, out.at[pl.ds(base, HBW)])
        cp1.wait()
        pltpu.sync_copy(buf1, out.at[pl.ds(base + HBW, HBW)])

    return sc_gather1


BLK = 1024
NBLK = B // BLK


def _tc_mlp_body(gu, gi, w1u, w1i, b1, w2t, b2, w3t, b3, wf, wfh, bf, out):
    f32 = jnp.float32
    b16 = jnp.bfloat16
    gur = gu[...]
    gir = gi[...]
    h1 = jnp.dot(gur.astype(b16), w1u[...], preferred_element_type=f32)
    h1 = h1 + jnp.dot(gir.astype(b16), w1i[...], preferred_element_type=f32)
    h1 = jnp.maximum(h1 + b1[...], 0.0)
    h2 = jnp.maximum(
        jnp.dot(h1.astype(b16), w2t[...], preferred_element_type=f32)
        + b2[...], 0.0)
    h3 = jnp.maximum(
        jnp.dot(h2.astype(b16), w3t[...], preferred_element_type=f32)
        + b3[...], 0.0)
    q = gur[:, :D] * gir[:, :D] * wf[:, :D]
    ql = q + h3 * wfh[...]
    out[...] = jnp.sum(ql, axis=1) + bf[0]


def _tc_mlp(gu, gi, w1u, w1i, b1, w2t, b2, w3t, b3, wf, wfh, bf):
    row_spec = pl.BlockSpec((BLK, 2 * D), lambda i: (i, 0))
    full = lambda shape: pl.BlockSpec(shape, lambda i: (0,) * len(shape))
    return pl.pallas_call(
        _tc_mlp_body,
        grid=(NBLK,),
        in_specs=[
            row_spec, row_spec,
            full((2 * D, 256)), full((2 * D, 256)), full((1, 256)),
            full((256, 128)), full((1, 128)),
            full((128, D)), full((1, D)),
            full((1, 2 * D)), full((1, D)), full((1,)),
        ],
        out_specs=pl.BlockSpec((BLK,), lambda i: (i,)),
        out_shape=jax.ShapeDtypeStruct((B,), jnp.float32),
    )(gu, gi, w1u, w1i, b1, w2t, b2, w3t, b3, wf, wfh, bf)


def kernel(gmf_user_table, gmf_item_table, mlp_user_table, mlp_item_table,
           W1, b1, W2, b2, W3, b3, Wf, bf, user_ids, item_ids):
    eye = jnp.eye(D, dtype=jnp.float32)
    zed = jnp.zeros((D, D), jnp.float32)
    eye_top = jnp.concatenate([eye, zed], axis=1)
    eye_bot = jnp.concatenate([zed, eye], axis=1)
    u_packed = _pack_pair(gmf_user_table.T, mlp_user_table.T, eye_top, eye_bot)
    i_packed = _pack_pair(gmf_item_table.T, mlp_item_table.T, eye_top, eye_bot)
    gather = _make_sc_gather1()
    gu = gather(u_packed, user_ids)
    gi = gather(i_packed, item_ids)
    z = jnp.zeros((D, 256), jnp.float32)
    w1u = jnp.concatenate([z, W1[:, :D].T], axis=0)
    w1i = jnp.concatenate([z, W1[:, D:].T], axis=0)
    wfm = jnp.concatenate(
        [Wf[:, :D], jnp.zeros((1, D), jnp.float32)], axis=1)
    bf16 = jnp.bfloat16
    return _tc_mlp(gu, gi, w1u.astype(bf16), w1i.astype(bf16),
                   b1.reshape(1, 256), W2.T.astype(bf16),
                   b2.reshape(1, 128), W3.T.astype(bf16),
                   b3.reshape(1, D), wfm, Wf[:, D:], bf)


# speedup vs baseline: 1.0218x; 1.0218x over previous
import functools

import jax
import jax.numpy as jnp
from jax import lax
from jax.experimental import pallas as pl
from jax.experimental.pallas import tpu as pltpu
from jax.experimental.pallas import tpu_sc as plsc

B = 16384
D = 64
NROWS = 100000

NC = 2
NS = 16
NW = NC * NS
BPW = B // NW


PACK_C = 12800
PACK_GRID = (NROWS + PACK_C - 1) // PACK_C


def _pack_body(gt, mt, eye_top, eye_bot, out):
    dn = (((0,), (0,)), ((), ()))
    out[...] = (
        lax.dot_general(gt[...], eye_top[...], dn,
                        preferred_element_type=jnp.float32)
        + lax.dot_general(mt[...], eye_bot[...], dn,
                          preferred_element_type=jnp.float32))


def _pack_pair(gt_T, mt_T, eye_top, eye_bot):
    in_spec = pl.BlockSpec((D, PACK_C), lambda i: (0, i))
    eye_spec = pl.BlockSpec((D, 2 * D), lambda i: (0, 0))
    return pl.pallas_call(
        _pack_body,
        grid=(PACK_GRID,),
        in_specs=[in_spec, in_spec, eye_spec, eye_spec],
        out_specs=pl.BlockSpec((PACK_C, 2 * D), lambda i: (i, 0)),
        out_shape=jax.ShapeDtypeStruct((NROWS, 2 * D), jnp.float32),
    )(gt_T, mt_T, eye_top, eye_bot)


HBW = BPW // 2


@functools.lru_cache(maxsize=1)
def _make_sc_gather1():
    mesh = plsc.VectorSubcoreMesh(
        core_axis_name="c", subcore_axis_name="s",
        num_cores=NC, num_subcores=NS)

    @functools.partial(
        pl.kernel,
        out_type=jax.ShapeDtypeStruct((B, 2 * D), jnp.float32),
        mesh=mesh,
        scratch_types=[
            pltpu.VMEM((HBW,), jnp.int32),
            pltpu.VMEM((HBW,), jnp.int32),
            pltpu.VMEM((HBW, 2 * D), jnp.float32),
            pltpu.VMEM((HBW, 2 * D), jnp.float32),
            pltpu.SemaphoreType.DMA,
            pltpu.SemaphoreType.DMA,
        ],
        compiler_params=pltpu.CompilerParams(use_tc_tiling_on_sc=True),
    )
    def sc_gather1(t_hbm, ids, out, idx0, idx1, buf0, buf1, sem0, sem1):
        wid = lax.axis_index("s") * NC + lax.axis_index("c")
        base = wid * BPW
        pltpu.sync_copy(ids.at[pl.ds(base, HBW)], idx0)
        pltpu.sync_copy(ids.at[pl.ds(base + HBW, HBW)], idx1)
        cp0 = pltpu.async_copy(t_hbm.at[idx0], buf0, sem0)
        cp1 = pltpu.async_copy(t_hbm.at[idx1], buf1, sem1)
        cp0.wait()
        pltpu.sync_copy(buf0, out.at[pl.ds(base, HBW)])
        cp1.wait()
        pltpu.sync_copy(buf1, out.at[pl.ds(base + HBW, HBW)])

    return sc_gather1


BLK = 1024
NBLK = B // BLK


def _tc_mlp_body(gu, gi, w1u, w1i, b1, w2t, b2, w3t, b3, wf, wfh, bf, out):
    f32 = jnp.float32
    b16 = jnp.bfloat16
    gur = gu[...]
    gir = gi[...]
    h1 = jnp.dot(gur.astype(b16), w1u[...], preferred_element_type=f32)
    h1 = h1 + jnp.dot(gir.astype(b16), w1i[...], preferred_element_type=f32)
    h1 = jnp.maximum(h1 + b1[...], 0.0)
    h2 = jnp.maximum(
        jnp.dot(h1.astype(b16), w2t[...], preferred_element_type=f32)
        + b2[...], 0.0)
    h3 = jnp.maximum(
        jnp.dot(h2.astype(b16), w3t[...], preferred_element_type=f32)
        + b3[...], 0.0)
    q = gur[:, :D] * gir[:, :D] * wf[:, :D]
    ql = q + h3 * wfh[...]
    out[...] = jnp.sum(ql, axis=1) + bf[0]


def _tc_mlp(gu, gi, w1u, w1i, b1, w2t, b2, w3t, b3, wf, wfh, bf):
    row_spec = pl.BlockSpec((BLK, 2 * D), lambda i: (i, 0))
    full = lambda shape: pl.BlockSpec(shape, lambda i: (0,) * len(shape))
    return pl.pallas_call(
        _tc_mlp_body,
        grid=(NBLK,),
        in_specs=[
            row_spec, row_spec,
            full((2 * D, 256)), full((2 * D, 256)), full((1, 256)),
            full((256, 128)), full((1, 128)),
            full((128, D)), full((1, D)),
            full((1, 2 * D)), full((1, D)), full((1,)),
        ],
        out_specs=pl.BlockSpec((BLK,), lambda i: (i,)),
        out_shape=jax.ShapeDtypeStruct((B,), jnp.float32),
    )(gu, gi, w1u, w1i, b1, w2t, b2, w3t, b3, wf, wfh, bf)


def kernel(gmf_user_table, gmf_item_table, mlp_user_table, mlp_item_table,
           W1, b1, W2, b2, W3, b3, Wf, bf, user_ids, item_ids):
    eye = jnp.eye(D, dtype=jnp.float32)
    zed = jnp.zeros((D, D), jnp.float32)
    eye_top = jnp.concatenate([eye, zed], axis=1)
    eye_bot = jnp.concatenate([zed, eye], axis=1)
    u_packed = _pack_pair(gmf_user_table.T, mlp_user_table.T, eye_top, eye_bot)
    i_packed = _pack_pair(gmf_item_table.T, mlp_item_table.T, eye_top, eye_bot)
    gather = _make_sc_gather1()
    gu = gather(u_packed, user_ids)
    gi = gather(i_packed, item_ids)
    z = jnp.zeros((D, 256), jnp.float32)
    w1u = jnp.concatenate([z, W1[:, :D].T], axis=0)
    w1i = jnp.concatenate([z, W1[:, D:].T], axis=0)
    wfm = jnp.concatenate(
        [Wf[:, :D], jnp.zeros((1, D), jnp.float32)], axis=1)
    bf16 = jnp.bfloat16
    return _tc_mlp(gu, gi, w1u.astype(bf16), w1i.astype(bf16),
                   b1.reshape(1, 256), W2.T.astype(bf16),
                   b2.reshape(1, 128), W3.T.astype(bf16),
                   b3.reshape(1, D), wfm, Wf[:, D:], bf)
